# baseline (speedup 1.0000x reference)
import jax
import jax.numpy as jnp
from jax.experimental import pallas as pl
from jax.experimental.pallas import tpu as pltpu


def _vq_argmin_kernel(x_hbm, c_hbm, out_ref, xbuf, cbuf, sem_x, sem_c):
    x_copy = pltpu.make_async_copy(x_hbm, xbuf, sem_x)
    x_copy.start()
    c_copy = pltpu.make_async_copy(c_hbm, cbuf, sem_c)
    c_copy.start()

    c_copy.wait()
    ct2 = cbuf[...].T * -2.0
    k = ct2.shape[1]
    cnorm = 0.25 * jnp.sum(ct2 * ct2, axis=0, keepdims=True)
    x_copy.wait()
    g2 = jnp.dot(xbuf[...], ct2, preferred_element_type=jnp.float32,
                 precision=jax.lax.Precision.HIGHEST)
    score = cnorm + g2
    m = jnp.min(score, axis=1, keepdims=True)
    col = jax.lax.broadcasted_iota(jnp.int32, score.shape, 1)
    idx = jnp.min(jnp.where(score == m, col, k), axis=1)
    out_ref[...] = idx


def kernel(x, centroids):
    n, d = x.shape
    k = centroids.shape[0]
    return pl.pallas_call(
        _vq_argmin_kernel,
        in_specs=[
            pl.BlockSpec(memory_space=pltpu.MemorySpace.HBM),
            pl.BlockSpec(memory_space=pltpu.MemorySpace.HBM),
        ],
        out_shape=jax.ShapeDtypeStruct((n,), jnp.int32),
        scratch_shapes=[
            pltpu.VMEM((n, d), jnp.float32),
            pltpu.VMEM((k, d), jnp.float32),
            pltpu.SemaphoreType.DMA,
            pltpu.SemaphoreType.DMA,
        ],
    )(x, centroids)
